# Initial kernel scaffold; baseline (speedup 1.0000x reference)
import jax
import jax.numpy as jnp
from jax.experimental import pallas as pl


def kernel(x, fact, inp, params, bias, msg_to, order, fact_type):
    raise NotImplementedError("write your pallas kernel here")


# speedup vs baseline: 1.4072x; 1.4072x over previous
import jax
import jax.numpy as jnp
from jax import lax
from jax.experimental import pallas as pl
from jax.experimental.pallas import tpu as pltpu
from jax.experimental.pallas import tpu_sc as plsc

NC = 2
NS = 16
L = 16
NW = NC * NS

F = 16384
H = 64
O = 64
FPW = F // NW
BLK = 64
NBLK = FPW // BLK
WB = 8
NWB = BLK // WB


def _sc_body(fact_hbm, xcol2_hbm, inp_hbm, params_hbm, bias_hbm, keep_hbm,
             out_hbm,
             xcol2_v, fact_v, inp_v, p_v, w_v, b_v, out_v, ids_v, keep_v,
             sem):
    c = lax.axis_index("c")
    s = lax.axis_index("s")
    wid = s * NC + c

    pltpu.sync_copy(xcol2_hbm, xcol2_v)
    pltpu.sync_copy(keep_hbm, keep_v)
    iota = lax.iota(jnp.int32, L)
    kb = [keep_v[k, :] for k in range(3)]
    ob = [1.0 - kb[k] for k in range(3)]
    zero_idx = jnp.zeros((L,), jnp.int32)

    def block_body(b, carry):
        base = wid * FPW + b * BLK
        pltpu.sync_copy(fact_hbm.at[pl.ds(base, BLK)], fact_v)
        for k in range(3):
            pltpu.sync_copy(inp_hbm.at[k, pl.ds(base, BLK)], inp_v.at[k])
        for cc in range(BLK // L):
            fidx = plsc.load_gather(fact_v, [iota + cc * L, zero_idx])
            ids_v[pl.ds(cc * L, L)] = plsc.load_gather(xcol2_v, [fidx])

        def prow(r, _):
            for cc in range(H // L):
                sl = pl.ds(cc * L, L)
                m0 = inp_v[0, r, sl] * kb[0] + ob[0]
                m1 = inp_v[1, r, sl] * kb[1] + ob[1]
                m2 = inp_v[2, r, sl] * kb[2] + ob[2]
                p_v[r, sl] = m0 * m1 * m2
            return 0

        lax.fori_loop(0, BLK, prow, 0)

        pltpu.async_copy(bias_hbm.at[ids_v], b_v, sem).wait()

        def batch_body(jb, _):
            isl = ids_v.at[pl.ds(jb * WB, WB)]
            pltpu.async_copy(params_hbm.at[isl], w_v, sem).wait()

            def fact_body(j, _):
                row = jb * WB + j
                row_idx = jnp.full((L,), 0, jnp.int32) + row
                accs = [b_v[row, pl.ds(cc * L, L)] for cc in range(O // L)]
                for h in range(H):
                    ph = plsc.load_gather(
                        p_v, [row_idx, jnp.full((L,), h, jnp.int32)])
                    for cc in range(O // L):
                        accs[cc] = accs[cc] + ph * w_v[
                            j, pl.ds(h * O + cc * L, L)]
                for cc in range(O // L):
                    out_v[row, pl.ds(cc * L, L)] = accs[cc]
                return 0

            lax.fori_loop(0, WB, fact_body, 0)
            return 0

        lax.fori_loop(0, NWB, batch_body, 0)
        pltpu.sync_copy(out_v, out_hbm.at[pl.ds(base, BLK)])
        return carry

    lax.fori_loop(0, NBLK, block_body, 0)


def kernel(x, fact, inp, params, bias, msg_to, order, fact_type):
    del fact_type
    idx3 = jnp.arange(3)
    keep = ((idx3 != msg_to) & (idx3 < order)).astype(jnp.float32)
    keep16 = jnp.broadcast_to(keep[:, None], (3, 16))
    xcol2 = x[:, 2].astype(jnp.int32)
    params2 = params.reshape(params.shape[0], H * O)
    bias2 = jnp.pad(bias.reshape(bias.shape[0], O), ((0, 0), (0, 128 - O)))

    mesh = plsc.VectorSubcoreMesh(core_axis_name="c", subcore_axis_name="s")
    run = pl.kernel(
        _sc_body,
        mesh=mesh,
        compiler_params=pltpu.CompilerParams(needs_layout_passes=False),
        out_type=jax.ShapeDtypeStruct((F, O), jnp.float32),
        scratch_types=[
            pltpu.VMEM((F,), jnp.int32),
            pltpu.VMEM((BLK, 2), jnp.int32),
            pltpu.VMEM((3, BLK, H), jnp.float32),
            pltpu.VMEM((BLK, H), jnp.float32),
            pltpu.VMEM((WB, H * O), jnp.float32),
            pltpu.VMEM((BLK, 128), jnp.float32),
            pltpu.VMEM((BLK, O), jnp.float32),
            pltpu.VMEM((BLK,), jnp.int32),
            pltpu.VMEM((3, 16), jnp.float32),
            pltpu.SemaphoreType.DMA,
        ],
    )
    return run(fact, xcol2, inp, params2, bias2, keep16)


# speedup vs baseline: 2.2163x; 1.5749x over previous
import jax
import jax.numpy as jnp
from jax import lax
from jax.experimental import pallas as pl
from jax.experimental.pallas import tpu as pltpu
from jax.experimental.pallas import tpu_sc as plsc

NC = 2
NS = 16
L = 16
NW = NC * NS

F = 16384
H = 64
O = 64
FPW = F // NW
BLK = 32
NBLK = FPW // BLK
WB = 16
NWB = BLK // WB


def _sc_body(fact_hbm, xcol2_hbm, inp_hbm, params_hbm, bias_hbm, keep_hbm,
             out_hbm,
             xcol2_v, fact_v, inp_v, p_v, w_v, b_v, out_v, ids_v, keep_v,
             semw0, semw1, semb):
    c = lax.axis_index("c")
    s = lax.axis_index("s")
    wid = s * NC + c

    pltpu.sync_copy(xcol2_hbm, xcol2_v)
    pltpu.sync_copy(keep_hbm, keep_v)
    iota = lax.iota(jnp.int32, L)
    kb = [keep_v[k, :] for k in range(3)]
    ob = [1.0 - kb[k] for k in range(3)]
    zero_idx = jnp.zeros((L,), jnp.int32)
    wsems = [semw0, semw1]

    def wdma(jb, parity):
        return pltpu.make_async_copy(
            params_hbm.at[ids_v.at[pl.ds(jb * WB, WB)]],
            w_v.at[parity], wsems[parity])

    def block_body(b, carry):
        base = wid * FPW + b * BLK
        pltpu.sync_copy(fact_hbm.at[pl.ds(base, BLK)], fact_v)
        for k in range(3):
            pltpu.sync_copy(inp_hbm.at[k, pl.ds(base, BLK)], inp_v.at[k])
        for cc in range(BLK // L):
            fidx = plsc.load_gather(fact_v, [iota + cc * L, zero_idx])
            ids_v[pl.ds(cc * L, L)] = plsc.load_gather(xcol2_v, [fidx])

        wdma(0, 0).start()
        bd = pltpu.make_async_copy(bias_hbm.at[ids_v], b_v, semb)
        bd.start()

        def prow(r, _):
            for cc in range(H // L):
                sl = pl.ds(cc * L, L)
                m0 = inp_v[0, r, sl] * kb[0] + ob[0]
                m1 = inp_v[1, r, sl] * kb[1] + ob[1]
                m2 = inp_v[2, r, sl] * kb[2] + ob[2]
                p_v[r, sl] = m0 * m1 * m2
            return 0

        lax.fori_loop(0, BLK, prow, 0)
        bd.wait()

        for jb in range(NWB):
            parity = jb % 2
            wdma(jb, parity).wait()
            if jb + 1 < NWB:
                wdma(jb + 1, 1 - parity).start()
            cur = w_v.at[parity]

            def fact_body(j, _, jb=jb, cur=cur):
                row = jb * WB + j
                row_idx = jnp.full((L,), 0, jnp.int32) + row
                accs = [b_v[row, pl.ds(cc * L, L)] for cc in range(O // L)]
                for h in range(H):
                    ph = plsc.load_gather(
                        p_v, [row_idx, jnp.full((L,), h, jnp.int32)])
                    for g in range(2):
                        w32 = cur[j, pl.ds(h * (O // 2) + g * L, L)]
                        lo, hi = plsc.unpack(
                            plsc.bitcast(w32, jnp.bfloat16),
                            format=plsc.PackFormat.INTERLEAVED)
                        accs[2 * g] = accs[2 * g] + ph * lo
                        accs[2 * g + 1] = accs[2 * g + 1] + ph * hi
                for cc in range(O // L):
                    out_v[row, pl.ds(cc * L, L)] = accs[cc]
                return 0

            lax.fori_loop(0, WB, fact_body, 0)

        pltpu.sync_copy(out_v, out_hbm.at[pl.ds(base, BLK)])
        return carry

    lax.fori_loop(0, NBLK, block_body, 0)


def kernel(x, fact, inp, params, bias, msg_to, order, fact_type):
    del fact_type
    idx3 = jnp.arange(3)
    keep = ((idx3 != msg_to) & (idx3 < order)).astype(jnp.float32)
    keep16 = jnp.broadcast_to(keep[:, None], (3, 16))
    xcol2 = x[:, 2].astype(jnp.int32)
    p_tab = lax.bitcast_convert_type(
        params.astype(jnp.bfloat16)
        .reshape(params.shape[0], H, 2, 2, 16)
        .transpose(0, 1, 2, 4, 3)
        .reshape(params.shape[0], H * O // 2, 2),
        jnp.int32)
    bias2 = jnp.pad(bias.reshape(bias.shape[0], O), ((0, 0), (0, 128 - O)))

    mesh = plsc.VectorSubcoreMesh(core_axis_name="c", subcore_axis_name="s")
    run = pl.kernel(
        _sc_body,
        mesh=mesh,
        compiler_params=pltpu.CompilerParams(needs_layout_passes=False),
        out_type=jax.ShapeDtypeStruct((F, O), jnp.float32),
        scratch_types=[
            pltpu.VMEM((F,), jnp.int32),
            pltpu.VMEM((BLK, 2), jnp.int32),
            pltpu.VMEM((3, BLK, H), jnp.float32),
            pltpu.VMEM((BLK, H), jnp.float32),
            pltpu.VMEM((2, WB, H * O // 2), jnp.int32),
            pltpu.VMEM((BLK, 128), jnp.float32),
            pltpu.VMEM((BLK, O), jnp.float32),
            pltpu.VMEM((BLK,), jnp.int32),
            pltpu.VMEM((3, 16), jnp.float32),
            pltpu.SemaphoreType.DMA,
            pltpu.SemaphoreType.DMA,
            pltpu.SemaphoreType.DMA,
        ],
    )
    return run(fact, xcol2, inp, p_tab, bias2, keep16)


# speedup vs baseline: 2.7992x; 1.2630x over previous
import jax
import jax.numpy as jnp
from jax import lax
from jax.experimental import pallas as pl
from jax.experimental.pallas import tpu as pltpu
from jax.experimental.pallas import tpu_sc as plsc

NC = 2
NS = 16
L = 16
NW = NC * NS

F = 16384
H = 64
O = 64
FPW = F // NW
BLK = 32
NBLK = FPW // BLK
WB = 16
NWB = BLK // WB


def _sc_body(fcol0_hbm, xcol2_hbm, inp_hbm, params_hbm, bias_hbm, keep_hbm,
             out_hbm,
             xcol2_v, fcol0_v, ids_v, inp_v, p_v, pp_v, b_v, w_v, out_v,
             keep_v, semw0, semw1, semb):
    c = lax.axis_index("c")
    s = lax.axis_index("s")
    wid = s * NC + c
    wbase = wid * FPW

    pltpu.sync_copy(xcol2_hbm, xcol2_v)
    pltpu.sync_copy(keep_hbm, keep_v)
    pltpu.sync_copy(fcol0_hbm.at[pl.ds(wbase, FPW)], fcol0_v)
    iota = lax.iota(jnp.int32, L)
    zero_idx = jnp.zeros((L,), jnp.int32)
    kb = [keep_v[k, :] for k in range(3)]
    ob = [1.0 - kb[k] for k in range(3)]

    def idchunk(cc, _):
        fidx = fcol0_v[pl.ds(pl.multiple_of(cc * L, L), L)]
        ids_v[pl.ds(pl.multiple_of(cc * L, L), L)] = plsc.load_gather(
            xcol2_v, [fidx])
        return 0

    lax.fori_loop(0, FPW // L, idchunk, 0)

    wsems = [semw0, semw1]

    def wdesc(gb, parity):
        st = pl.multiple_of(gb * WB, WB)
        return pltpu.make_async_copy(
            params_hbm.at[ids_v.at[pl.ds(st, WB)]],
            w_v.at[parity], wsems[parity])

    wdesc(0, 0).start()

    def block_body(b, carry):
        base = wbase + b * BLK
        for k in range(3):
            pltpu.sync_copy(inp_hbm.at[k, pl.ds(base, BLK)], inp_v.at[k])
        bd = pltpu.make_async_copy(
            bias_hbm.at[ids_v.at[pl.ds(pl.multiple_of(b * BLK, BLK), BLK)]],
            b_v, semb)
        bd.start()

        def prow(r, _):
            for cc in range(H // L):
                sl = pl.ds(cc * L, L)
                m0 = inp_v[0, r, sl] * kb[0] + ob[0]
                m1 = inp_v[1, r, sl] * kb[1] + ob[1]
                m2 = inp_v[2, r, sl] * kb[2] + ob[2]
                p_v[r, sl] = m0 * m1 * m2
            row_idx = jnp.full((L,), 0, jnp.int32) + r
            for half in range(2):
                ae = plsc.load_gather(p_v, [row_idx, 2 * iota + 32 * half])
                ao = plsc.load_gather(p_v, [row_idx, 2 * iota + 1 + 32 * half])
                pk = plsc.pack(ae, ao, format=plsc.PackFormat.INTERLEAVED)
                pp_v[r, pl.ds(half * L, L)] = plsc.bitcast(pk, jnp.int32)
            return 0

        lax.fori_loop(0, BLK, prow, 0)
        bd.wait()

        for jb in range(NWB):
            gb = 2 * b + jb
            wdesc(gb, jb).wait()

            @pl.when(gb + 1 < 2 * NBLK)
            def _():
                wdesc(gb + 1, 1 - jb).start()

            cur = w_v.at[jb]

            def fact_body(j, _, jb=jb, cur=cur):
                row = jb * WB + j
                row_idx = jnp.full((L,), 0, jnp.int32) + row
                accs = [b_v[row, pl.ds(cc * L, L)] for cc in range(O // L)]
                zerop = jnp.zeros((2 * L,), jnp.bfloat16)
                for hq in range(4):
                    accp = [zerop] * (O // L)
                    for t in range(8):
                        h2 = hq * 8 + t
                        pp = plsc.load_gather(
                            pp_v, [row_idx, jnp.full((L,), h2, jnp.int32)])
                        ppb = plsc.bitcast(pp, jnp.bfloat16)
                        for cc in range(O // L):
                            w32 = cur[j, pl.ds(h2 * O + cc * L, L)]
                            accp[cc] = accp[cc] + plsc.bitcast(
                                w32, jnp.bfloat16) * ppb
                    for cc in range(O // L):
                        lo, hi = plsc.unpack(
                            accp[cc], format=plsc.PackFormat.INTERLEAVED)
                        accs[cc] = accs[cc] + lo
                        accs[cc] = accs[cc] + hi
                for cc in range(O // L):
                    out_v[row, pl.ds(cc * L, L)] = accs[cc]
                return 0

            lax.fori_loop(0, WB, fact_body, 0)

        pltpu.sync_copy(out_v, out_hbm.at[pl.ds(base, BLK)])
        return carry

    lax.fori_loop(0, NBLK, block_body, 0)


def kernel(x, fact, inp, params, bias, msg_to, order, fact_type):
    del fact_type
    idx3 = jnp.arange(3)
    keep = ((idx3 != msg_to) & (idx3 < order)).astype(jnp.float32)
    keep16 = jnp.broadcast_to(keep[:, None], (3, 16))
    xcol2 = x[:, 2].astype(jnp.int32)
    fcol0 = fact[:, 0].astype(jnp.int32)
    p_tab = lax.bitcast_convert_type(
        params.astype(jnp.bfloat16)
        .reshape(params.shape[0], H // 2, 2, O)
        .transpose(0, 1, 3, 2),
        jnp.int32).reshape(params.shape[0], H * O // 2)
    bias2 = jnp.pad(bias.reshape(bias.shape[0], O), ((0, 0), (0, 128 - O)))

    mesh = plsc.VectorSubcoreMesh(core_axis_name="c", subcore_axis_name="s")
    run = pl.kernel(
        _sc_body,
        mesh=mesh,
        compiler_params=pltpu.CompilerParams(needs_layout_passes=False),
        out_type=jax.ShapeDtypeStruct((F, O), jnp.float32),
        scratch_types=[
            pltpu.VMEM((F,), jnp.int32),
            pltpu.VMEM((FPW,), jnp.int32),
            pltpu.VMEM((FPW,), jnp.int32),
            pltpu.VMEM((3, BLK, H), jnp.float32),
            pltpu.VMEM((BLK, H), jnp.float32),
            pltpu.VMEM((BLK, H // 2), jnp.int32),
            pltpu.VMEM((BLK, 128), jnp.float32),
            pltpu.VMEM((2, WB, H * O // 2), jnp.int32),
            pltpu.VMEM((BLK, O), jnp.float32),
            pltpu.VMEM((3, 16), jnp.float32),
            pltpu.SemaphoreType.DMA,
            pltpu.SemaphoreType.DMA,
            pltpu.SemaphoreType.DMA,
        ],
    )
    return run(fcol0, xcol2, inp, p_tab, bias2, keep16)


# speedup vs baseline: 2.8549x; 1.0199x over previous
import jax
import jax.numpy as jnp
from jax import lax
from jax.experimental import pallas as pl
from jax.experimental.pallas import tpu as pltpu
from jax.experimental.pallas import tpu_sc as plsc

NC = 2
NS = 16
L = 16
NW = NC * NS

F = 16384
H = 64
O = 64
FPW = F // NW
BLK = 32
NBLK = FPW // BLK
WB = 16
NWB = BLK // WB


def _sc_body(fcol0_hbm, xcol2_hbm, inp_hbm, params_hbm, bias_hbm, keep_hbm,
             out_hbm,
             xcol2_v, fcol0_v, ids_v, inp_v, p_v, pp_v, b_v, w_v, out_v,
             keep_v, semw0, semw1, semb):
    c = lax.axis_index("c")
    s = lax.axis_index("s")
    wid = s * NC + c
    wbase = wid * FPW

    pltpu.sync_copy(xcol2_hbm, xcol2_v)
    pltpu.sync_copy(keep_hbm, keep_v)
    pltpu.sync_copy(fcol0_hbm.at[pl.ds(wbase, FPW)], fcol0_v)
    iota = lax.iota(jnp.int32, L)
    zero_idx = jnp.zeros((L,), jnp.int32)
    kb = [keep_v[k, :] for k in range(3)]
    ob = [1.0 - kb[k] for k in range(3)]

    def idchunk(cc, _):
        fidx = fcol0_v[pl.ds(pl.multiple_of(cc * L, L), L)]
        ids_v[pl.ds(pl.multiple_of(cc * L, L), L)] = plsc.load_gather(
            xcol2_v, [fidx])
        return 0

    lax.fori_loop(0, FPW // L, idchunk, 0)

    wsems = [semw0, semw1]

    def wdesc(gb, parity):
        st = pl.multiple_of(gb * WB, WB)
        return pltpu.make_async_copy(
            params_hbm.at[ids_v.at[pl.ds(st, WB)]],
            w_v.at[parity], wsems[parity])

    wdesc(0, 0).start()

    def block_body(b, carry):
        base = wbase + b * BLK
        for k in range(3):
            pltpu.sync_copy(inp_hbm.at[k, pl.ds(base, BLK)], inp_v.at[k])
        bd = pltpu.make_async_copy(
            bias_hbm.at[ids_v.at[pl.ds(pl.multiple_of(b * BLK, BLK), BLK)]],
            b_v, semb)
        bd.start()

        def prow(r, _):
            for cc in range(H // L):
                sl = pl.ds(cc * L, L)
                m0 = inp_v[0, r, sl] * kb[0] + ob[0]
                m1 = inp_v[1, r, sl] * kb[1] + ob[1]
                m2 = inp_v[2, r, sl] * kb[2] + ob[2]
                p_v[r, sl] = m0 * m1 * m2
            row_idx = jnp.full((L,), 0, jnp.int32) + r
            for half in range(2):
                ae = plsc.load_gather(p_v, [row_idx, 2 * iota + 32 * half])
                ao = plsc.load_gather(p_v, [row_idx, 2 * iota + 1 + 32 * half])
                pk = plsc.pack(ae, ao, format=plsc.PackFormat.INTERLEAVED)
                pp_v[r, pl.ds(half * L, L)] = plsc.bitcast(pk, jnp.int32)
            return 0

        lax.fori_loop(0, BLK, prow, 0)
        bd.wait()

        for jb in range(NWB):
            gb = 2 * b + jb
            wdesc(gb, jb).wait()

            @pl.when(gb + 1 < 2 * NBLK)
            def _():
                wdesc(gb + 1, 1 - jb).start()

            cur = w_v.at[jb]

            def fact_body(j, _, jb=jb, cur=cur):
                row = jb * WB + j
                row_idx = jnp.full((L,), 0, jnp.int32) + row
                accs = [b_v[row, pl.ds(cc * L, L)] for cc in range(O // L)]
                zerop = jnp.zeros((2 * L,), jnp.bfloat16)
                for hq in range(4):
                    accp = [zerop] * (O // L)
                    for t in range(8):
                        h2 = hq * 8 + t
                        pp = plsc.load_gather(
                            pp_v, [row_idx, jnp.full((L,), h2, jnp.int32)])
                        ppb = plsc.bitcast(pp, jnp.bfloat16)
                        for cc in range(O // L):
                            w32 = cur[j, pl.ds(h2 * O + cc * L, L)]
                            accp[cc] = accp[cc] + plsc.bitcast(
                                w32, jnp.bfloat16) * ppb
                    for cc in range(O // L):
                        lo, hi = plsc.unpack(
                            accp[cc], format=plsc.PackFormat.INTERLEAVED)
                        accs[cc] = accs[cc] + lo
                        accs[cc] = accs[cc] + hi
                for cc in range(O // L):
                    out_v[row, pl.ds(cc * L, L)] = accs[cc]
                return 0

            lax.fori_loop(0, WB, fact_body, 0)

        pltpu.sync_copy(out_v, out_hbm.at[pl.ds(base, BLK)])
        return carry

    lax.fori_loop(0, NBLK, block_body, 0)


def kernel(x, fact, inp, params, bias, msg_to, order, fact_type):
    del fact_type
    idx3 = jnp.arange(3)
    keep = ((idx3 != msg_to) & (idx3 < order)).astype(jnp.float32)
    keep16 = jnp.broadcast_to(keep[:, None], (3, 16))
    xcol2 = x[:, 2].astype(jnp.int32)
    fcol0 = fact[:, 0].astype(jnp.int32)
    ev = lax.bitcast_convert_type(
        params[:, 0::2, :].astype(jnp.bfloat16), jnp.uint16
    ).astype(jnp.uint32)
    od = lax.bitcast_convert_type(
        params[:, 1::2, :].astype(jnp.bfloat16), jnp.uint16
    ).astype(jnp.uint32)
    p_tab = lax.bitcast_convert_type(
        ev | (od << 16), jnp.int32).reshape(params.shape[0], H * O // 2)
    bias2 = jnp.pad(bias.reshape(bias.shape[0], O), ((0, 0), (0, 128 - O)))

    mesh = plsc.VectorSubcoreMesh(core_axis_name="c", subcore_axis_name="s")
    run = pl.kernel(
        _sc_body,
        mesh=mesh,
        compiler_params=pltpu.CompilerParams(needs_layout_passes=False),
        out_type=jax.ShapeDtypeStruct((F, O), jnp.float32),
        scratch_types=[
            pltpu.VMEM((F,), jnp.int32),
            pltpu.VMEM((FPW,), jnp.int32),
            pltpu.VMEM((FPW,), jnp.int32),
            pltpu.VMEM((3, BLK, H), jnp.float32),
            pltpu.VMEM((BLK, H), jnp.float32),
            pltpu.VMEM((BLK, H // 2), jnp.int32),
            pltpu.VMEM((BLK, 128), jnp.float32),
            pltpu.VMEM((2, WB, H * O // 2), jnp.int32),
            pltpu.VMEM((BLK, O), jnp.float32),
            pltpu.VMEM((3, 16), jnp.float32),
            pltpu.SemaphoreType.DMA,
            pltpu.SemaphoreType.DMA,
            pltpu.SemaphoreType.DMA,
        ],
    )
    return run(fcol0, xcol2, inp, p_tab, bias2, keep16)


# speedup vs baseline: 3.1160x; 1.0915x over previous
import jax
import jax.numpy as jnp
from jax import lax
from jax.experimental import pallas as pl
from jax.experimental.pallas import tpu as pltpu
from jax.experimental.pallas import tpu_sc as plsc

NC = 2
NS = 16
L = 16
NW = NC * NS

F = 16384
H = 64
O = 64
FPW = F // NW
SB = 128
NSB = FPW // SB
SUB = 32
NSUB = SB // SUB
WB = 8
NWB = SUB // WB
NGB = FPW // WB


def _sc_body(fcol0_hbm, xcol2_hbm, inp_hbm, params_hbm, bias_hbm, keep_hbm,
             out_hbm,
             xcol2_v, fcol0_v, ids_v, inp_v, pt_v, ppf_v, b_v, w_v, out_v,
             keep_v, semw0, semw1, semb):
    c = lax.axis_index("c")
    s = lax.axis_index("s")
    wid = s * NC + c
    wbase = wid * FPW

    pltpu.sync_copy(xcol2_hbm, xcol2_v)
    pltpu.sync_copy(keep_hbm, keep_v)
    pltpu.sync_copy(fcol0_hbm.at[pl.ds(wbase, FPW)], fcol0_v)
    iota = lax.iota(jnp.int32, L)
    kb = [keep_v[k, :] for k in range(3)]
    ob = [1.0 - kb[k] for k in range(3)]

    def idchunk(cc, _):
        fidx = fcol0_v[pl.ds(pl.multiple_of(cc * L, L), L)]
        ids_v[pl.ds(pl.multiple_of(cc * L, L), L)] = plsc.load_gather(
            xcol2_v, [fidx])
        return 0

    lax.fori_loop(0, FPW // L, idchunk, 0)

    wsems = [semw0, semw1]

    def wdesc(gb, parity):
        st = pl.multiple_of(gb * WB, WB)
        return pltpu.make_async_copy(
            params_hbm.at[ids_v.at[pl.ds(st, WB)]],
            w_v.at[parity], wsems[parity])

    wdesc(0, 0).start()

    def sb_body(sbi, carry):
        base = wbase + sbi * SB
        for k in range(3):
            pltpu.sync_copy(inp_hbm.at[k, :, pl.ds(base, SB)], inp_v.at[k])

        def prod_h(h, _):
            for fc in range(SB // L):
                sl = pl.ds(fc * L, L)
                m0 = inp_v[0, h, sl] * kb[0] + ob[0]
                m1 = inp_v[1, h, sl] * kb[1] + ob[1]
                m2 = inp_v[2, h, sl] * kb[2] + ob[2]
                pt_v[h, sl] = m0 * m1 * m2
            return 0

        lax.fori_loop(0, H, prod_h, 0)

        def pair_f(f, _):
            ri = jnp.full((L,), 0, jnp.int32) + f
            for half in range(2):
                ae = plsc.load_gather(pt_v, [2 * iota + 32 * half, ri])
                ao = plsc.load_gather(pt_v, [2 * iota + 1 + 32 * half, ri])
                pk = plsc.pack(ae, ao, format=plsc.PackFormat.INTERLEAVED)
                ppf_v[pl.ds(pl.multiple_of(f * 32 + half * L, L), L)] = (
                    plsc.bitcast(pk, jnp.int32))
            return 0

        lax.fori_loop(0, SB, pair_f, 0)

        def sub_body(sub, _):
            soff = sbi * SB + sub * SUB
            bd = pltpu.make_async_copy(
                bias_hbm.at[ids_v.at[pl.ds(pl.multiple_of(soff, SUB), SUB)]],
                b_v, semb)
            bd.start()
            sgb = sbi * (SB // WB) + sub * NWB

            for jb in range(NWB):
                gb = sgb + jb
                parity = jb % 2
                wdesc(gb, parity).wait()

                @pl.when(gb + 1 < NGB)
                def _():
                    wdesc(gb + 1, 1 - parity).start()

                if jb == 0:
                    bd.wait()
                cur = w_v.at[parity]

                def fact_body(j, _, jb=jb, cur=cur):
                    row = sub * SUB + jb * WB + j
                    fb = row * 32
                    row_idx = jnp.full((L,), 0, jnp.int32)
                    accs = [b_v[jb * WB + j, pl.ds(cc * L, L)]
                            for cc in range(O // L)]
                    zerop = jnp.zeros((2 * L,), jnp.bfloat16)
                    for hq in range(4):
                        accp = [zerop] * (O // L)
                        for t in range(8):
                            h2 = hq * 8 + t
                            pp = plsc.load_gather(ppf_v, [row_idx + (fb + h2)])
                            ppb = plsc.bitcast(pp, jnp.bfloat16)
                            for cc in range(O // L):
                                w32 = cur[j, pl.ds(h2 * O + cc * L, L)]
                                accp[cc] = accp[cc] + plsc.bitcast(
                                    w32, jnp.bfloat16) * ppb
                        for cc in range(O // L):
                            lo, hi = plsc.unpack(
                                accp[cc], format=plsc.PackFormat.INTERLEAVED)
                            accs[cc] = accs[cc] + lo
                            accs[cc] = accs[cc] + hi
                    for cc in range(O // L):
                        plsc.store_scatter(
                            out_v, [cc * L + iota, row_idx + row], accs[cc])
                    return 0

                lax.fori_loop(0, WB, fact_body, 0)
            return 0

        lax.fori_loop(0, NSUB, sub_body, 0)
        pltpu.sync_copy(out_v, out_hbm.at[:, pl.ds(base, SB)])
        return carry

    lax.fori_loop(0, NSB, sb_body, 0)


def kernel(x, fact, inp, params, bias, msg_to, order, fact_type):
    del fact_type
    idx3 = jnp.arange(3)
    keep = ((idx3 != msg_to) & (idx3 < order)).astype(jnp.float32)
    keep16 = jnp.broadcast_to(keep[:, None], (3, 16))
    xcol2 = x.T[2].astype(jnp.int32)
    fcol0 = fact.T[0].astype(jnp.int32)
    inp_t = jnp.transpose(inp, (0, 2, 1))
    ev = lax.bitcast_convert_type(
        params[:, 0::2, :].astype(jnp.bfloat16), jnp.uint16
    ).astype(jnp.uint32)
    od = lax.bitcast_convert_type(
        params[:, 1::2, :].astype(jnp.bfloat16), jnp.uint16
    ).astype(jnp.uint32)
    p_tab = lax.bitcast_convert_type(
        ev | (od << 16), jnp.int32).reshape(params.shape[0], H * O // 2)
    bias2 = jnp.pad(bias.reshape(bias.shape[0], O), ((0, 0), (0, 128 - O)))

    mesh = plsc.VectorSubcoreMesh(core_axis_name="c", subcore_axis_name="s")
    run = pl.kernel(
        _sc_body,
        mesh=mesh,
        compiler_params=pltpu.CompilerParams(needs_layout_passes=False),
        out_type=jax.ShapeDtypeStruct((O, F), jnp.float32),
        scratch_types=[
            pltpu.VMEM((F,), jnp.int32),
            pltpu.VMEM((FPW,), jnp.int32),
            pltpu.VMEM((FPW,), jnp.int32),
            pltpu.VMEM((3, H, SB), jnp.float32),
            pltpu.VMEM((H, SB), jnp.float32),
            pltpu.VMEM((SB * 32,), jnp.int32),
            pltpu.VMEM((SUB, 128), jnp.float32),
            pltpu.VMEM((2, WB, H * O // 2), jnp.int32),
            pltpu.VMEM((O, SB), jnp.float32),
            pltpu.VMEM((3, 16), jnp.float32),
            pltpu.SemaphoreType.DMA,
            pltpu.SemaphoreType.DMA,
            pltpu.SemaphoreType.DMA,
        ],
    )
    out_t = run(fcol0, xcol2, inp_t, p_tab, bias2, keep16)
    return jnp.transpose(out_t)
